# Initial kernel scaffold; baseline (speedup 1.0000x reference)
import jax
import jax.numpy as jnp
from jax.experimental import pallas as pl


def kernel(x, table_0, table_1, table_2, table_3, table_4, table_5, table_6, table_7, table_8, table_9, table_10, table_11, table_12, table_13, table_14, table_15, table_16, table_17, table_18, table_19, table_20, table_21, table_22, table_23, table_24, table_25):
    raise NotImplementedError("write your pallas kernel here")


# speedup vs baseline: 2.5159x; 2.5159x over previous
import functools

import jax
import jax.numpy as jnp
from jax import lax
from jax.experimental import pallas as pl
from jax.experimental.pallas import tpu as pltpu
from jax.experimental.pallas import tpu_sc as plsc

_NUM_FEATURES = 26
_EMB = 16
_BATCH = 4096
_SEQ = 20
_BS = _BATCH * _SEQ
_NW = 32
_CHUNK = _BS // _NW


def _emb_body(xt, *rest):
    tables = rest[:_NUM_FEATURES]
    out = rest[_NUM_FEATURES]
    idx_a, idx_b, rows_a, rows_b, gsem_a, gsem_b, wsem = rest[_NUM_FEATURES + 1:]
    idx_bufs = (idx_a, idx_b)
    row_bufs = (rows_a, rows_b)
    gsems = (gsem_a, gsem_b)

    wid = lax.axis_index("s") * 2 + lax.axis_index("c")
    base = wid * _CHUNK

    pltpu.sync_copy(xt.at[0, pl.ds(base, _CHUNK)], idx_bufs[0])
    cur_gather = pltpu.async_copy(tables[0].at[idx_bufs[0]], row_bufs[0], gsems[0])

    pending_write = None
    next_gather = None
    for f in range(_NUM_FEATURES):
        cur = f % 2
        nxt = (f + 1) % 2
        if f + 1 < _NUM_FEATURES:
            pltpu.sync_copy(xt.at[f + 1, pl.ds(base, _CHUNK)], idx_bufs[nxt])
            if pending_write is not None:
                pending_write.wait()
                pending_write = None
            next_gather = pltpu.async_copy(
                tables[f + 1].at[idx_bufs[nxt]], row_bufs[nxt], gsems[nxt])
        cur_gather.wait()
        if pending_write is not None:
            pending_write.wait()
        pending_write = pltpu.async_copy(
            row_bufs[cur], out.at[pl.ds(base, _CHUNK), f], wsem)
        cur_gather = next_gather
    pending_write.wait()


@functools.partial(
    pl.kernel,
    out_type=jax.ShapeDtypeStruct((_BS, _NUM_FEATURES, _EMB), jnp.float32),
    mesh=plsc.VectorSubcoreMesh(core_axis_name="c", subcore_axis_name="s"),
    scratch_types=[
        pltpu.VMEM((_CHUNK,), jnp.int32),
        pltpu.VMEM((_CHUNK,), jnp.int32),
        pltpu.VMEM((_CHUNK, _EMB), jnp.float32),
        pltpu.VMEM((_CHUNK, _EMB), jnp.float32),
        pltpu.SemaphoreType.DMA,
        pltpu.SemaphoreType.DMA,
        pltpu.SemaphoreType.DMA,
    ],
    compiler_params=pltpu.CompilerParams(use_tc_tiling_on_sc=False),
)
def _emb_call(*args):
    _emb_body(*args)


def kernel(x, table_0, table_1, table_2, table_3, table_4, table_5, table_6,
           table_7, table_8, table_9, table_10, table_11, table_12, table_13,
           table_14, table_15, table_16, table_17, table_18, table_19,
           table_20, table_21, table_22, table_23, table_24, table_25):
    tables = (table_0, table_1, table_2, table_3, table_4, table_5, table_6,
              table_7, table_8, table_9, table_10, table_11, table_12,
              table_13, table_14, table_15, table_16, table_17, table_18,
              table_19, table_20, table_21, table_22, table_23, table_24,
              table_25)
    xt = x.reshape(_BS, _NUM_FEATURES).astype(jnp.int32).T
    out = _emb_call(xt, *tables)
    return out.reshape(_BATCH, _SEQ, _NUM_FEATURES * _EMB)
